# Initial kernel scaffold; baseline (speedup 1.0000x reference)
import jax
import jax.numpy as jnp
from jax.experimental import pallas as pl


def kernel(x, edge_index, batch, c1_w1, c1_b1, c1_w2, c1_b2, c2_w1, c2_b1, c2_w2, c2_b2, c3_w1, c3_b1, c3_w2, c3_b2, l1_w, l1_b, l2_w, l2_b):
    raise NotImplementedError("write your pallas kernel here")


# speedup vs baseline: 33.1455x; 33.1455x over previous
import functools

import jax
import jax.numpy as jnp
from jax import lax
from jax.experimental import pallas as pl
from jax.experimental.pallas import tpu as pltpu
from jax.experimental.pallas import tpu_sc as plsc

N = 100000
E = 3200000
G = 1000
F = 16
NC = 2
NS = 16
NW = NC * NS
EPT = E // NW
CHUNK = 1000
NCH = EPT // CHUNK
RPT = N // NS
NODE_CH = N // CHUNK


def _seg_body(h_hbm, src_hbm, dst_hbm, out_hbm, idx_s, idx_d, rows, acc, sem):
    c = lax.axis_index("c")
    s = lax.axis_index("s")

    @pl.when(c == 1)
    def _():
        zero = jnp.zeros((F,), jnp.float32)

        def zb(i, carry):
            rows[i, :] = zero
            return carry

        lax.fori_loop(0, CHUNK, zb, 0)

    for k in range(NODE_CH):
        @pl.when(s == (k % NS))
        def _(k=k):
            off = k * CHUNK

            @pl.when(c == 0)
            def _():
                pltpu.sync_copy(h_hbm.at[pl.ds(off, CHUNK)], acc.at[pl.ds(off, CHUNK)])

            @pl.when(c == 1)
            def _():
                pltpu.sync_copy(rows, acc.at[pl.ds(off, CHUNK)])

    plsc.subcore_barrier()

    base = (s * NC + c) * EPT

    def body(i, carry):
        off = base + i * CHUNK
        pltpu.sync_copy(src_hbm.at[pl.ds(off, CHUNK)], idx_s)
        pltpu.sync_copy(dst_hbm.at[pl.ds(off, CHUNK)], idx_d)
        pltpu.async_copy(h_hbm.at[idx_s], rows, sem).wait()
        pltpu.sync_copy(rows, acc.at[idx_d], add=True)
        return carry

    lax.fori_loop(0, NCH, body, 0)
    plsc.subcore_barrier()

    for k in range(NODE_CH):
        @pl.when(s == (k % NS))
        def _(k=k):
            off = k * CHUNK
            pltpu.sync_copy(acc.at[pl.ds(off, CHUNK)], out_hbm.at[c, pl.ds(off, CHUNK)])


_segsum = pl.kernel(
    _seg_body,
    out_type=jax.ShapeDtypeStruct((NC, N, F), jnp.float32),
    mesh=plsc.VectorSubcoreMesh(core_axis_name="c", subcore_axis_name="s"),
    scratch_types=[
        pltpu.VMEM((CHUNK,), jnp.int32),
        pltpu.VMEM((CHUNK,), jnp.int32),
        pltpu.VMEM((CHUNK, F), jnp.float32),
        pltpu.VMEM_SHARED((N, F), jnp.float32),
        pltpu.SemaphoreType.DMA,
    ],
    compiler_params=pltpu.CompilerParams(use_tc_tiling_on_sc=False),
)


def _pool_body(h_hbm, batch_hbm, out_hbm, idx_b, rows, accg, sem):
    c = lax.axis_index("c")
    s = lax.axis_index("s")
    wid = s * NC + c

    @pl.when(s == 0)
    def _():
        zero = jnp.zeros((F,), jnp.float32)

        def zb(i, carry):
            rows[i, :] = zero
            return carry

        lax.fori_loop(0, G, zb, 0)
        pltpu.sync_copy(rows.at[pl.ds(0, G)], accg)

    plsc.subcore_barrier()

    for k in range(NODE_CH):
        @pl.when(wid == (k % NW))
        def _(k=k):
            off = k * CHUNK
            pltpu.sync_copy(batch_hbm.at[pl.ds(off, CHUNK)], idx_b)
            pltpu.sync_copy(h_hbm.at[pl.ds(off, CHUNK)], rows)
            pltpu.sync_copy(rows, accg.at[idx_b], add=True)

    plsc.subcore_barrier()

    @pl.when(s == 0)
    def _():
        pltpu.sync_copy(accg, out_hbm.at[c])


_pool = pl.kernel(
    _pool_body,
    out_type=jax.ShapeDtypeStruct((NC, G, F), jnp.float32),
    mesh=plsc.VectorSubcoreMesh(core_axis_name="c", subcore_axis_name="s"),
    scratch_types=[
        pltpu.VMEM((CHUNK,), jnp.int32),
        pltpu.VMEM((CHUNK, F), jnp.float32),
        pltpu.VMEM_SHARED((G, F), jnp.float32),
        pltpu.SemaphoreType.DMA,
    ],
    compiler_params=pltpu.CompilerParams(use_tc_tiling_on_sc=False),
)


BN = 4000


def _mlp_body(a_ref, w1_ref, b1_ref, w2_ref, b2_ref, o_ref):
    a = a_ref[0] + a_ref[1]
    t = jnp.maximum(jnp.dot(a, w1_ref[...], preferred_element_type=jnp.float32) + b1_ref[...], 0.0)
    o_ref[...] = jnp.maximum(jnp.dot(t, w2_ref[...], preferred_element_type=jnp.float32) + b2_ref[...], 0.0)


def _mlp(agg2, w1, b1, w2, b2):
    grid = (N // BN,)
    return pl.pallas_call(
        _mlp_body,
        grid=grid,
        in_specs=[
            pl.BlockSpec((NC, BN, F), lambda i: (0, i, 0)),
            pl.BlockSpec((F, F), lambda i: (0, 0)),
            pl.BlockSpec((1, F), lambda i: (0, 0)),
            pl.BlockSpec((F, F), lambda i: (0, 0)),
            pl.BlockSpec((1, F), lambda i: (0, 0)),
        ],
        out_specs=pl.BlockSpec((BN, F), lambda i: (i, 0)),
        out_shape=jax.ShapeDtypeStruct((N, F), jnp.float32),
    )(agg2, w1, b1, w2, b2)


def _head_body(p_ref, l1w_ref, l1b_ref, l2w_ref, l2b_ref, o_ref):
    p = p_ref[0] + p_ref[1]
    t = jnp.maximum(jnp.dot(p, l1w_ref[...], preferred_element_type=jnp.float32) + l1b_ref[...], 0.0)
    o_ref[...] = jnp.dot(t, l2w_ref[...], preferred_element_type=jnp.float32) + l2b_ref[...]


def _head(pooled2, l1w, l1b, l2w, l2b):
    return pl.pallas_call(
        _head_body,
        out_shape=jax.ShapeDtypeStruct((G, 1), jnp.float32),
    )(pooled2, l1w, l1b, l2w, l2b)


def kernel(x, edge_index, batch,
           c1_w1, c1_b1, c1_w2, c1_b2,
           c2_w1, c2_b1, c2_w2, c2_b2,
           c3_w1, c3_b1, c3_w2, c3_b2,
           l1_w, l1_b, l2_w, l2_b):
    xp = jnp.pad(x, ((0, 0), (0, F - x.shape[1])))
    w1p = jnp.pad(c1_w1, ((0, F - c1_w1.shape[0]), (0, 0)))
    src = edge_index[0]
    dst = edge_index[1]

    h = xp
    for w1, b1, w2, b2 in (
        (w1p, c1_b1, c1_w2, c1_b2),
        (c2_w1, c2_b1, c2_w2, c2_b2),
        (c3_w1, c3_b1, c3_w2, c3_b2),
    ):
        agg2 = _segsum(h, src, dst)
        h = _mlp(agg2, w1, b1.reshape(1, F), w2, b2.reshape(1, F))

    pooled2 = _pool(h, batch)
    return _head(pooled2, l1_w, l1_b.reshape(1, F), l2_w, l2_b.reshape(1, 1))


# speedup vs baseline: 34.4019x; 1.0379x over previous
import functools

import jax
import jax.numpy as jnp
from jax import lax
from jax.experimental import pallas as pl
from jax.experimental.pallas import tpu as pltpu
from jax.experimental.pallas import tpu_sc as plsc

N = 100000
E = 3200000
G = 1000
F = 16
NC = 2
NS = 16
NW = NC * NS
EPT = E // NW
CHUNK = 800
NCH = EPT // CHUNK
NPAIR = (NCH - 1) // 2
POOL_CHUNK = 1000
NODE_CH = N // POOL_CHUNK


def _seg_body(h_hbm, src_hbm, dst_hbm, out_hbm,
              idx_sa, idx_da, rows_a, idx_sb, idx_db, rows_b,
              acc, sem_ga, sem_gb, sem_sa, sem_sb):
    c = lax.axis_index("c")
    s = lax.axis_index("s")

    @pl.when(c == 1)
    def _():
        zero = jnp.zeros((F,), jnp.float32)

        def zb(i, carry):
            rows_a[i, :] = zero
            return carry

        lax.fori_loop(0, CHUNK, zb, 0)

    for k in range(NODE_CH):
        @pl.when(s == (k % NS))
        def _(k=k):
            off = k * POOL_CHUNK

            @pl.when(c == 0)
            def _():
                pltpu.sync_copy(h_hbm.at[pl.ds(off, POOL_CHUNK)], acc.at[pl.ds(off, POOL_CHUNK)])

            @pl.when(c == 1)
            def _():
                pltpu.sync_copy(rows_a, acc.at[pl.ds(off, CHUNK)])
                pltpu.sync_copy(rows_a.at[pl.ds(0, POOL_CHUNK - CHUNK)],
                                acc.at[pl.ds(off + CHUNK, POOL_CHUNK - CHUNK)])

    plsc.subcore_barrier()

    base = (s * NC + c) * EPT

    def load_idx(off, idx_s, idx_d):
        pltpu.sync_copy(src_hbm.at[pl.ds(off, CHUNK)], idx_s)
        pltpu.sync_copy(dst_hbm.at[pl.ds(off, CHUNK)], idx_d)

    def gather_start(idx_s, rows, sem):
        pltpu.async_copy(h_hbm.at[idx_s], rows, sem)

    def gather_wait(idx_s, rows, sem):
        pltpu.make_async_copy(h_hbm.at[idx_s], rows, sem).wait()

    def scat_start(rows, idx_d, sem):
        pltpu.async_copy(rows, acc.at[idx_d], sem, add=True)

    def scat_wait(rows, sem):
        pltpu.make_async_copy(h_hbm.at[pl.ds(0, CHUNK)], rows, sem).wait()

    load_idx(base, idx_sa, idx_da)
    gather_start(idx_sa, rows_a, sem_ga)

    def body(i, carry):
        e_off = base + (2 * i) * CHUNK
        o_off = e_off + CHUNK
        gather_wait(idx_sa, rows_a, sem_ga)
        scat_start(rows_a, idx_da, sem_sa)

        @pl.when(i > 0)
        def _():
            scat_wait(rows_b, sem_sb)

        load_idx(o_off, idx_sb, idx_db)
        gather_start(idx_sb, rows_b, sem_gb)
        gather_wait(idx_sb, rows_b, sem_gb)
        scat_start(rows_b, idx_db, sem_sb)
        scat_wait(rows_a, sem_sa)

        @pl.when(i < NPAIR - 1)
        def _():
            load_idx(e_off + 2 * CHUNK, idx_sa, idx_da)
            gather_start(idx_sa, rows_a, sem_ga)

        return carry

    lax.fori_loop(0, NPAIR, body, 0)

    load_idx(base + (NCH - 1) * CHUNK, idx_sa, idx_da)
    gather_start(idx_sa, rows_a, sem_ga)
    gather_wait(idx_sa, rows_a, sem_ga)
    scat_wait(rows_b, sem_sb)
    scat_start(rows_a, idx_da, sem_sa)
    scat_wait(rows_a, sem_sa)

    plsc.subcore_barrier()

    for k in range(NODE_CH):
        @pl.when(s == (k % NS))
        def _(k=k):
            off = k * POOL_CHUNK
            pltpu.sync_copy(acc.at[pl.ds(off, POOL_CHUNK)], out_hbm.at[c, pl.ds(off, POOL_CHUNK)])


_segsum = pl.kernel(
    _seg_body,
    out_type=jax.ShapeDtypeStruct((NC, N, F), jnp.float32),
    mesh=plsc.VectorSubcoreMesh(core_axis_name="c", subcore_axis_name="s"),
    scratch_types=[
        pltpu.VMEM((CHUNK,), jnp.int32),
        pltpu.VMEM((CHUNK,), jnp.int32),
        pltpu.VMEM((CHUNK, F), jnp.float32),
        pltpu.VMEM((CHUNK,), jnp.int32),
        pltpu.VMEM((CHUNK,), jnp.int32),
        pltpu.VMEM((CHUNK, F), jnp.float32),
        pltpu.VMEM_SHARED((N, F), jnp.float32),
        pltpu.SemaphoreType.DMA,
        pltpu.SemaphoreType.DMA,
        pltpu.SemaphoreType.DMA,
        pltpu.SemaphoreType.DMA,
    ],
    compiler_params=pltpu.CompilerParams(use_tc_tiling_on_sc=False),
)


def _pool_body(h_hbm, batch_hbm, out_hbm, idx_b, rows, accg, sem):
    c = lax.axis_index("c")
    s = lax.axis_index("s")
    wid = s * NC + c

    @pl.when(s == 0)
    def _():
        zero = jnp.zeros((F,), jnp.float32)

        def zb(i, carry):
            rows[i, :] = zero
            return carry

        lax.fori_loop(0, G, zb, 0)
        pltpu.sync_copy(rows.at[pl.ds(0, G)], accg)

    plsc.subcore_barrier()

    for k in range(NODE_CH):
        @pl.when(wid == (k % NW))
        def _(k=k):
            off = k * POOL_CHUNK
            pltpu.sync_copy(batch_hbm.at[pl.ds(off, POOL_CHUNK)], idx_b)
            pltpu.sync_copy(h_hbm.at[pl.ds(off, POOL_CHUNK)], rows)
            pltpu.sync_copy(rows, accg.at[idx_b], add=True)

    plsc.subcore_barrier()

    @pl.when(s == 0)
    def _():
        pltpu.sync_copy(accg, out_hbm.at[c])


_pool = pl.kernel(
    _pool_body,
    out_type=jax.ShapeDtypeStruct((NC, G, F), jnp.float32),
    mesh=plsc.VectorSubcoreMesh(core_axis_name="c", subcore_axis_name="s"),
    scratch_types=[
        pltpu.VMEM((POOL_CHUNK,), jnp.int32),
        pltpu.VMEM((POOL_CHUNK, F), jnp.float32),
        pltpu.VMEM_SHARED((G, F), jnp.float32),
        pltpu.SemaphoreType.DMA,
    ],
    compiler_params=pltpu.CompilerParams(use_tc_tiling_on_sc=False),
)


BN = 4000


def _mlp_body(a_ref, w1_ref, b1_ref, w2_ref, b2_ref, o_ref):
    a = a_ref[0] + a_ref[1]
    t = jnp.maximum(jnp.dot(a, w1_ref[...], preferred_element_type=jnp.float32) + b1_ref[...], 0.0)
    o_ref[...] = jnp.maximum(jnp.dot(t, w2_ref[...], preferred_element_type=jnp.float32) + b2_ref[...], 0.0)


def _mlp(agg2, w1, b1, w2, b2):
    grid = (N // BN,)
    return pl.pallas_call(
        _mlp_body,
        grid=grid,
        in_specs=[
            pl.BlockSpec((NC, BN, F), lambda i: (0, i, 0)),
            pl.BlockSpec((F, F), lambda i: (0, 0)),
            pl.BlockSpec((1, F), lambda i: (0, 0)),
            pl.BlockSpec((F, F), lambda i: (0, 0)),
            pl.BlockSpec((1, F), lambda i: (0, 0)),
        ],
        out_specs=pl.BlockSpec((BN, F), lambda i: (i, 0)),
        out_shape=jax.ShapeDtypeStruct((N, F), jnp.float32),
    )(agg2, w1, b1, w2, b2)


def _head_body(p_ref, l1w_ref, l1b_ref, l2w_ref, l2b_ref, o_ref):
    p = p_ref[0] + p_ref[1]
    t = jnp.maximum(jnp.dot(p, l1w_ref[...], preferred_element_type=jnp.float32) + l1b_ref[...], 0.0)
    o_ref[...] = jnp.dot(t, l2w_ref[...], preferred_element_type=jnp.float32) + l2b_ref[...]


def _head(pooled2, l1w, l1b, l2w, l2b):
    return pl.pallas_call(
        _head_body,
        out_shape=jax.ShapeDtypeStruct((G, 1), jnp.float32),
    )(pooled2, l1w, l1b, l2w, l2b)


def kernel(x, edge_index, batch,
           c1_w1, c1_b1, c1_w2, c1_b2,
           c2_w1, c2_b1, c2_w2, c2_b2,
           c3_w1, c3_b1, c3_w2, c3_b2,
           l1_w, l1_b, l2_w, l2_b):
    xp = jnp.pad(x, ((0, 0), (0, F - x.shape[1])))
    w1p = jnp.pad(c1_w1, ((0, F - c1_w1.shape[0]), (0, 0)))
    src = edge_index[0]
    dst = edge_index[1]

    h = xp
    for w1, b1, w2, b2 in (
        (w1p, c1_b1, c1_w2, c1_b2),
        (c2_w1, c2_b1, c2_w2, c2_b2),
        (c3_w1, c3_b1, c3_w2, c3_b2),
    ):
        agg2 = _segsum(h, src, dst)
        h = _mlp(agg2, w1, b1.reshape(1, F), w2, b2.reshape(1, F))

    pooled2 = _pool(h, batch)
    return _head(pooled2, l1_w, l1_b.reshape(1, F), l2_w, l2_b.reshape(1, 1))


# speedup vs baseline: 45.0380x; 1.3092x over previous
import functools

import jax
import jax.numpy as jnp
from jax import lax
from jax.experimental import pallas as pl
from jax.experimental.pallas import tpu as pltpu
from jax.experimental.pallas import tpu_sc as plsc

N = 100000
E = 3200000
G = 1000
F = 16
NC = 2
NS = 16
NW = NC * NS
EPT = E // NW
CHUNK = 800
NCH = EPT // CHUNK
NPAIR = (NCH - 1) // 2
POOL_CHUNK = 1000
NODE_CH = N // POOL_CHUNK


def _seg_body(h_hbm, src_hbm, dst_hbm, out_hbm,
              idx_sa, idx_da, rows_a, idx_sb, idx_db, rows_b,
              acc, sem_ga, sem_gb, sem_sa, sem_sb, sem_ia, sem_ib):
    c = lax.axis_index("c")
    s = lax.axis_index("s")

    @pl.when(c == 1)
    def _():
        zero = jnp.zeros((F,), jnp.float32)

        def zb(i, carry):
            rows_a[i, :] = zero
            return carry

        lax.fori_loop(0, CHUNK, zb, 0)

    for k in range(NODE_CH):
        @pl.when(s == (k % NS))
        def _(k=k):
            off = k * POOL_CHUNK

            @pl.when(c == 0)
            def _():
                pltpu.sync_copy(h_hbm.at[pl.ds(off, POOL_CHUNK)], acc.at[pl.ds(off, POOL_CHUNK)])

            @pl.when(c == 1)
            def _():
                pltpu.sync_copy(rows_a, acc.at[pl.ds(off, CHUNK)])
                pltpu.sync_copy(rows_a.at[pl.ds(0, POOL_CHUNK - CHUNK)],
                                acc.at[pl.ds(off + CHUNK, POOL_CHUNK - CHUNK)])

    plsc.subcore_barrier()

    base = (s * NC + c) * EPT

    def load_idx(off, idx_s, idx_d):
        pltpu.sync_copy(src_hbm.at[pl.ds(off, CHUNK)], idx_s)
        pltpu.sync_copy(dst_hbm.at[pl.ds(off, CHUNK)], idx_d)

    def idx_start(off, idx_s, idx_d, sem):
        pltpu.async_copy(src_hbm.at[pl.ds(off, CHUNK)], idx_s, sem)
        pltpu.async_copy(dst_hbm.at[pl.ds(off, CHUNK)], idx_d, sem)

    def idx_wait(idx_s, idx_d, sem):
        pltpu.make_async_copy(src_hbm.at[pl.ds(0, CHUNK)], idx_s, sem).wait()
        pltpu.make_async_copy(dst_hbm.at[pl.ds(0, CHUNK)], idx_d, sem).wait()

    def gather_start(idx_s, rows, sem):
        pltpu.async_copy(h_hbm.at[idx_s], rows, sem)

    def gather_wait(idx_s, rows, sem):
        pltpu.make_async_copy(h_hbm.at[idx_s], rows, sem).wait()

    def scat_start(rows, idx_d, sem):
        pltpu.async_copy(rows, acc.at[idx_d], sem, add=True)

    def scat_wait(rows, sem):
        pltpu.make_async_copy(h_hbm.at[pl.ds(0, CHUNK)], rows, sem).wait()

    load_idx(base, idx_sa, idx_da)
    gather_start(idx_sa, rows_a, sem_ga)

    def body(i, carry):
        e_off = base + (2 * i) * CHUNK
        o_off = e_off + CHUNK

        @pl.when(i > 0)
        def _():
            scat_wait(rows_b, sem_sb)

        idx_start(o_off, idx_sb, idx_db, sem_ib)
        gather_wait(idx_sa, rows_a, sem_ga)
        scat_start(rows_a, idx_da, sem_sa)
        idx_wait(idx_sb, idx_db, sem_ib)
        gather_start(idx_sb, rows_b, sem_gb)

        scat_wait(rows_a, sem_sa)

        @pl.when(i < NPAIR - 1)
        def _():
            idx_start(e_off + 2 * CHUNK, idx_sa, idx_da, sem_ia)

        gather_wait(idx_sb, rows_b, sem_gb)
        scat_start(rows_b, idx_db, sem_sb)

        @pl.when(i < NPAIR - 1)
        def _():
            idx_wait(idx_sa, idx_da, sem_ia)
            gather_start(idx_sa, rows_a, sem_ga)

        return carry

    lax.fori_loop(0, NPAIR, body, 0)

    load_idx(base + (NCH - 1) * CHUNK, idx_sa, idx_da)
    gather_start(idx_sa, rows_a, sem_ga)
    gather_wait(idx_sa, rows_a, sem_ga)
    scat_wait(rows_b, sem_sb)
    scat_start(rows_a, idx_da, sem_sa)
    scat_wait(rows_a, sem_sa)

    plsc.subcore_barrier()

    for k in range(NODE_CH):
        @pl.when(s == (k % NS))
        def _(k=k):
            off = k * POOL_CHUNK
            pltpu.sync_copy(acc.at[pl.ds(off, POOL_CHUNK)], out_hbm.at[c, pl.ds(off, POOL_CHUNK)])


_segsum = pl.kernel(
    _seg_body,
    out_type=jax.ShapeDtypeStruct((NC, N, F), jnp.float32),
    mesh=plsc.VectorSubcoreMesh(core_axis_name="c", subcore_axis_name="s"),
    scratch_types=[
        pltpu.VMEM((CHUNK,), jnp.int32),
        pltpu.VMEM((CHUNK,), jnp.int32),
        pltpu.VMEM((CHUNK, F), jnp.float32),
        pltpu.VMEM((CHUNK,), jnp.int32),
        pltpu.VMEM((CHUNK,), jnp.int32),
        pltpu.VMEM((CHUNK, F), jnp.float32),
        pltpu.VMEM_SHARED((N, F), jnp.float32),
        pltpu.SemaphoreType.DMA,
        pltpu.SemaphoreType.DMA,
        pltpu.SemaphoreType.DMA,
        pltpu.SemaphoreType.DMA,
        pltpu.SemaphoreType.DMA,
        pltpu.SemaphoreType.DMA,
    ],
    compiler_params=pltpu.CompilerParams(use_tc_tiling_on_sc=False),
)


def _pool_body(h_hbm, batch_hbm, out_hbm, idx_b, rows, accg, sem):
    c = lax.axis_index("c")
    s = lax.axis_index("s")
    wid = s * NC + c

    @pl.when(s == 0)
    def _():
        zero = jnp.zeros((F,), jnp.float32)

        def zb(i, carry):
            rows[i, :] = zero
            return carry

        lax.fori_loop(0, G, zb, 0)
        pltpu.sync_copy(rows.at[pl.ds(0, G)], accg)

    plsc.subcore_barrier()

    for k in range(NODE_CH):
        @pl.when(wid == (k % NW))
        def _(k=k):
            off = k * POOL_CHUNK
            pltpu.sync_copy(batch_hbm.at[pl.ds(off, POOL_CHUNK)], idx_b)
            pltpu.sync_copy(h_hbm.at[pl.ds(off, POOL_CHUNK)], rows)
            pltpu.sync_copy(rows, accg.at[idx_b], add=True)

    plsc.subcore_barrier()

    @pl.when(s == 0)
    def _():
        pltpu.sync_copy(accg, out_hbm.at[c])


_pool = pl.kernel(
    _pool_body,
    out_type=jax.ShapeDtypeStruct((NC, G, F), jnp.float32),
    mesh=plsc.VectorSubcoreMesh(core_axis_name="c", subcore_axis_name="s"),
    scratch_types=[
        pltpu.VMEM((POOL_CHUNK,), jnp.int32),
        pltpu.VMEM((POOL_CHUNK, F), jnp.float32),
        pltpu.VMEM_SHARED((G, F), jnp.float32),
        pltpu.SemaphoreType.DMA,
    ],
    compiler_params=pltpu.CompilerParams(use_tc_tiling_on_sc=False),
)


BN = 4000


def _mlp_body(a_ref, w1_ref, b1_ref, w2_ref, b2_ref, o_ref):
    a = a_ref[0] + a_ref[1]
    t = jnp.maximum(jnp.dot(a, w1_ref[...], preferred_element_type=jnp.float32) + b1_ref[...], 0.0)
    o_ref[...] = jnp.maximum(jnp.dot(t, w2_ref[...], preferred_element_type=jnp.float32) + b2_ref[...], 0.0)


def _mlp(agg2, w1, b1, w2, b2):
    grid = (N // BN,)
    return pl.pallas_call(
        _mlp_body,
        grid=grid,
        in_specs=[
            pl.BlockSpec((NC, BN, F), lambda i: (0, i, 0)),
            pl.BlockSpec((F, F), lambda i: (0, 0)),
            pl.BlockSpec((1, F), lambda i: (0, 0)),
            pl.BlockSpec((F, F), lambda i: (0, 0)),
            pl.BlockSpec((1, F), lambda i: (0, 0)),
        ],
        out_specs=pl.BlockSpec((BN, F), lambda i: (i, 0)),
        out_shape=jax.ShapeDtypeStruct((N, F), jnp.float32),
    )(agg2, w1, b1, w2, b2)


def _head_body(p_ref, l1w_ref, l1b_ref, l2w_ref, l2b_ref, o_ref):
    p = p_ref[0] + p_ref[1]
    t = jnp.maximum(jnp.dot(p, l1w_ref[...], preferred_element_type=jnp.float32) + l1b_ref[...], 0.0)
    o_ref[...] = jnp.dot(t, l2w_ref[...], preferred_element_type=jnp.float32) + l2b_ref[...]


def _head(pooled2, l1w, l1b, l2w, l2b):
    return pl.pallas_call(
        _head_body,
        out_shape=jax.ShapeDtypeStruct((G, 1), jnp.float32),
    )(pooled2, l1w, l1b, l2w, l2b)


def kernel(x, edge_index, batch,
           c1_w1, c1_b1, c1_w2, c1_b2,
           c2_w1, c2_b1, c2_w2, c2_b2,
           c3_w1, c3_b1, c3_w2, c3_b2,
           l1_w, l1_b, l2_w, l2_b):
    xp = jnp.pad(x, ((0, 0), (0, F - x.shape[1])))
    w1p = jnp.pad(c1_w1, ((0, F - c1_w1.shape[0]), (0, 0)))
    src = edge_index[0]
    dst = edge_index[1]

    h = xp
    for w1, b1, w2, b2 in (
        (w1p, c1_b1, c1_w2, c1_b2),
        (c2_w1, c2_b1, c2_w2, c2_b2),
        (c3_w1, c3_b1, c3_w2, c3_b2),
    ):
        agg2 = _segsum(h, src, dst)
        h = _mlp(agg2, w1, b1.reshape(1, F), w2, b2.reshape(1, F))

    pooled2 = _pool(h, batch)
    return _head(pooled2, l1_w, l1_b.reshape(1, F), l2_w, l2_b.reshape(1, 1))


# speedup vs baseline: 67.2007x; 1.4921x over previous
import functools

import jax
import jax.numpy as jnp
from jax import lax
from jax.experimental import pallas as pl
from jax.experimental.pallas import tpu as pltpu
from jax.experimental.pallas import tpu_sc as plsc

N = 100000
E = 3200000
G = 1000
F = 16
NC = 2
NS = 16
NW = NC * NS
EPT = E // NW
CHUNK = 800
NCH = EPT // CHUNK
NPAIR = (NCH - 1) // 2
POOL_CHUNK = 1000
NODE_CH = N // POOL_CHUNK

NP = 100032
AR = NP * F // 128
GP = 1024
GR = GP * F // 128


def _seg_body(h_hbm, ei_hbm, out_hbm,
              idx_sa, idx_da, rows_a, idx_sb, idx_db, rows_b,
              acc, sem_ga, sem_gb, sem_sa, sem_sb, sem_ia, sem_ib):
    c = lax.axis_index("c")
    s = lax.axis_index("s")

    @pl.when(c == 1)
    def _():
        zero = jnp.zeros((F,), jnp.float32)

        def zb(i, carry):
            rows_a[i, :] = zero
            return carry

        lax.fori_loop(0, CHUNK, zb, 0)

    for k in range(NODE_CH):
        @pl.when(s == (k % NS))
        def _(k=k):
            off = k * POOL_CHUNK

            @pl.when(c == 0)
            def _():
                pltpu.sync_copy(h_hbm.at[pl.ds(off, POOL_CHUNK)], acc.at[pl.ds(off, POOL_CHUNK)])

            @pl.when(c == 1)
            def _():
                pltpu.sync_copy(rows_a, acc.at[pl.ds(off, CHUNK)])
                pltpu.sync_copy(rows_a.at[pl.ds(0, POOL_CHUNK - CHUNK)],
                                acc.at[pl.ds(off + CHUNK, POOL_CHUNK - CHUNK)])

    plsc.subcore_barrier()

    base = (s * NC + c) * EPT

    def load_idx(off, idx_s, idx_d):
        pltpu.sync_copy(ei_hbm.at[0, pl.ds(off, CHUNK)], idx_s)
        pltpu.sync_copy(ei_hbm.at[1, pl.ds(off, CHUNK)], idx_d)

    def idx_start(off, idx_s, idx_d, sem):
        pltpu.async_copy(ei_hbm.at[0, pl.ds(off, CHUNK)], idx_s, sem)
        pltpu.async_copy(ei_hbm.at[1, pl.ds(off, CHUNK)], idx_d, sem)

    def idx_wait(idx_s, idx_d, sem):
        pltpu.make_async_copy(ei_hbm.at[0, pl.ds(0, CHUNK)], idx_s, sem).wait()
        pltpu.make_async_copy(ei_hbm.at[1, pl.ds(0, CHUNK)], idx_d, sem).wait()

    def gather_start(idx_s, rows, sem):
        pltpu.async_copy(h_hbm.at[idx_s], rows, sem)

    def gather_wait(idx_s, rows, sem):
        pltpu.make_async_copy(h_hbm.at[idx_s], rows, sem).wait()

    def scat_start(rows, idx_d, sem):
        pltpu.async_copy(rows, acc.at[idx_d], sem, add=True)

    def scat_wait(rows, sem):
        pltpu.make_async_copy(h_hbm.at[pl.ds(0, CHUNK)], rows, sem).wait()

    load_idx(base, idx_sa, idx_da)
    gather_start(idx_sa, rows_a, sem_ga)

    def body(i, carry):
        e_off = base + (2 * i) * CHUNK
        o_off = e_off + CHUNK

        @pl.when(i > 0)
        def _():
            scat_wait(rows_b, sem_sb)

        idx_start(o_off, idx_sb, idx_db, sem_ib)
        gather_wait(idx_sa, rows_a, sem_ga)
        scat_start(rows_a, idx_da, sem_sa)
        idx_wait(idx_sb, idx_db, sem_ib)
        gather_start(idx_sb, rows_b, sem_gb)

        scat_wait(rows_a, sem_sa)

        @pl.when(i < NPAIR - 1)
        def _():
            idx_start(e_off + 2 * CHUNK, idx_sa, idx_da, sem_ia)

        gather_wait(idx_sb, rows_b, sem_gb)
        scat_start(rows_b, idx_db, sem_sb)

        @pl.when(i < NPAIR - 1)
        def _():
            idx_wait(idx_sa, idx_da, sem_ia)
            gather_start(idx_sa, rows_a, sem_ga)

        return carry

    lax.fori_loop(0, NPAIR, body, 0)

    load_idx(base + (NCH - 1) * CHUNK, idx_sa, idx_da)
    gather_start(idx_sa, rows_a, sem_ga)
    gather_wait(idx_sa, rows_a, sem_ga)
    scat_wait(rows_b, sem_sb)
    scat_start(rows_a, idx_da, sem_sa)
    scat_wait(rows_a, sem_sa)

    plsc.subcore_barrier()

    for k in range(NODE_CH):
        @pl.when(s == (k % NS))
        def _(k=k):
            off = k * POOL_CHUNK
            pltpu.sync_copy(acc.at[pl.ds(off, POOL_CHUNK)], out_hbm.at[c, pl.ds(off, POOL_CHUNK)])


_segsum = pl.kernel(
    _seg_body,
    out_type=jax.ShapeDtypeStruct((NC, NP, F), jnp.float32),
    mesh=plsc.VectorSubcoreMesh(core_axis_name="c", subcore_axis_name="s"),
    scratch_types=[
        pltpu.VMEM((CHUNK,), jnp.int32),
        pltpu.VMEM((CHUNK,), jnp.int32),
        pltpu.VMEM((CHUNK, F), jnp.float32),
        pltpu.VMEM((CHUNK,), jnp.int32),
        pltpu.VMEM((CHUNK,), jnp.int32),
        pltpu.VMEM((CHUNK, F), jnp.float32),
        pltpu.VMEM_SHARED((NP, F), jnp.float32),
        pltpu.SemaphoreType.DMA,
        pltpu.SemaphoreType.DMA,
        pltpu.SemaphoreType.DMA,
        pltpu.SemaphoreType.DMA,
        pltpu.SemaphoreType.DMA,
        pltpu.SemaphoreType.DMA,
    ],
    compiler_params=pltpu.CompilerParams(use_tc_tiling_on_sc=False),
)


def _pool_body(h_hbm, batch_hbm, out_hbm, idx_b, rows, accg, sem):
    c = lax.axis_index("c")
    s = lax.axis_index("s")
    wid = s * NC + c

    @pl.when(s == 0)
    def _():
        zero = jnp.zeros((F,), jnp.float32)

        def zb(i, carry):
            rows[i, :] = zero
            return carry

        lax.fori_loop(0, POOL_CHUNK, zb, 0)
        pltpu.sync_copy(rows, accg.at[pl.ds(0, POOL_CHUNK)])
        pltpu.sync_copy(rows.at[pl.ds(0, GP - POOL_CHUNK)],
                        accg.at[pl.ds(POOL_CHUNK, GP - POOL_CHUNK)])

    plsc.subcore_barrier()

    for k in range(NODE_CH):
        @pl.when(wid == (k % NW))
        def _(k=k):
            off = k * POOL_CHUNK
            pltpu.sync_copy(batch_hbm.at[pl.ds(off, POOL_CHUNK)], idx_b)
            pltpu.sync_copy(h_hbm.at[pl.ds(off, POOL_CHUNK)], rows)
            pltpu.sync_copy(rows, accg.at[idx_b], add=True)

    plsc.subcore_barrier()

    @pl.when(s == 0)
    def _():
        pltpu.sync_copy(accg, out_hbm.at[c])


_pool = pl.kernel(
    _pool_body,
    out_type=jax.ShapeDtypeStruct((NC, GP, F), jnp.float32),
    mesh=plsc.VectorSubcoreMesh(core_axis_name="c", subcore_axis_name="s"),
    scratch_types=[
        pltpu.VMEM((POOL_CHUNK,), jnp.int32),
        pltpu.VMEM((POOL_CHUNK, F), jnp.float32),
        pltpu.VMEM_SHARED((GP, F), jnp.float32),
        pltpu.SemaphoreType.DMA,
    ],
    compiler_params=pltpu.CompilerParams(use_tc_tiling_on_sc=False),
)


BR = 4168


def _mlp_body(a_ref, w1_ref, b1_ref, w2_ref, b2_ref, o_ref):
    a = a_ref[0] + a_ref[1]
    t = jnp.maximum(jnp.dot(a, w1_ref[...], preferred_element_type=jnp.float32) + b1_ref[...], 0.0)
    o_ref[...] = jnp.maximum(jnp.dot(t, w2_ref[...], preferred_element_type=jnp.float32) + b2_ref[...], 0.0)


def _mlp(agg2, w1t, b1t, w2t, b2t):
    grid = (AR // BR,)
    return pl.pallas_call(
        _mlp_body,
        grid=grid,
        in_specs=[
            pl.BlockSpec((NC, BR, 128), lambda i: (0, i, 0)),
            pl.BlockSpec((128, 128), lambda i: (0, 0)),
            pl.BlockSpec((1, 128), lambda i: (0, 0)),
            pl.BlockSpec((128, 128), lambda i: (0, 0)),
            pl.BlockSpec((1, 128), lambda i: (0, 0)),
        ],
        out_specs=pl.BlockSpec((BR, 128), lambda i: (i, 0)),
        out_shape=jax.ShapeDtypeStruct((AR, 128), jnp.float32),
    )(agg2, w1t, b1t, w2t, b2t)


def _head_body(p_ref, l1w_ref, l1b_ref, l2w_ref, l2b_ref, o_ref):
    p = p_ref[0] + p_ref[1]
    t = jnp.maximum(jnp.dot(p, l1w_ref[...], preferred_element_type=jnp.float32) + l1b_ref[...], 0.0)
    o_ref[...] = jnp.dot(t, l2w_ref[...], preferred_element_type=jnp.float32) + l2b_ref[...]


def _head(pooled2, l1wt, l1bt, l2wt, l2bt):
    return pl.pallas_call(
        _head_body,
        out_shape=jax.ShapeDtypeStruct((GR, 8), jnp.float32),
    )(pooled2, l1wt, l1bt, l2wt, l2bt)


def _blockdiag(w):
    return jnp.kron(jnp.eye(8, dtype=w.dtype), w)


def kernel(x, edge_index, batch,
           c1_w1, c1_b1, c1_w2, c1_b2,
           c2_w1, c2_b1, c2_w2, c2_b2,
           c3_w1, c3_b1, c3_w2, c3_b2,
           l1_w, l1_b, l2_w, l2_b):
    xp = jnp.pad(x, ((0, NP - N), (0, F - x.shape[1])))
    w1p = jnp.pad(c1_w1, ((0, F - c1_w1.shape[0]), (0, 0)))

    h = xp
    for w1, b1, w2, b2 in (
        (w1p, c1_b1, c1_w2, c1_b2),
        (c2_w1, c2_b1, c2_w2, c2_b2),
        (c3_w1, c3_b1, c3_w2, c3_b2),
    ):
        agg2 = _segsum(h, edge_index)
        h128 = _mlp(agg2.reshape(NC, AR, 128),
                    _blockdiag(w1), jnp.tile(b1, 8).reshape(1, 128),
                    _blockdiag(w2), jnp.tile(b2, 8).reshape(1, 128))
        h = h128.reshape(NP, F)

    pooled2 = _pool(h, batch)
    out8 = _head(pooled2.reshape(NC, GR, 128),
                 _blockdiag(l1_w), jnp.tile(l1_b, 8).reshape(1, 128),
                 _blockdiag(l2_w), jnp.tile(l2_b, 8).reshape(1, 8))
    return out8.reshape(GP, 1)[:G]
